# baseline (speedup 1.0000x reference)
import functools

import jax
import jax.numpy as jnp
from jax import lax
from jax.experimental import pallas as pl
from jax.experimental.pallas import tpu as pltpu
from jax.experimental.pallas import tpu_sc as plsc

VOCAB = 1000000
EMBED = 32
BATCH = 16384
SEQ_LEN = 200

NUM_CORES = 2
NUM_SUBCORES = 16
NUM_WORKERS = NUM_CORES * NUM_SUBCORES
ROWS_PER_WORKER = BATCH // NUM_WORKERS
CHUNK = 8
NCHUNKS = ROWS_PER_WORKER // CHUNK
SPLITS = ((0, 104), (104, 96))
GATHERS = len(SPLITS) * CHUNK


TK = 512
TBLOCKS = 999936 // TK
TAIL = VOCAB - TBLOCKS * TK
TPW = -(-TBLOCKS // NUM_WORKERS)


def _tp_body(tt_hbm, tail_hbm, tl_hbm, in_v, out_v, ip0, ip1, op0, op1):
    cid = lax.axis_index("c")
    sid = lax.axis_index("s")
    wid = sid * NUM_CORES + cid
    ipsems = (ip0, ip1)
    opsems = (op0, op1)
    row_lo = jnp.arange(16, dtype=jnp.int32)
    row_hi = row_lo + 16

    def blkid(i):
        return i * NUM_WORKERS + wid

    def start_in(i, b):
        def go():
            pltpu.make_async_copy(
                tt_hbm.at[:, pl.ds(blkid(i) * TK, TK)], in_v.at[b], ipsems[b],
            ).start()
        pl.when(blkid(i) < TBLOCKS)(go)

    def wait_in(i, b):
        def go():
            pltpu.make_async_copy(
                tt_hbm.at[:, pl.ds(0, TK)], in_v.at[b], ipsems[b],
            ).wait()
        pl.when(blkid(i) < TBLOCKS)(go)

    def start_out(i, b):
        def go():
            pltpu.make_async_copy(
                out_v.at[b],
                tl_hbm.at[pl.ds(blkid(i) * TK * EMBED, TK * EMBED)],
                opsems[b],
            ).start()
        pl.when(blkid(i) < TBLOCKS)(go)

    def wait_out(i, b):
        def go():
            pltpu.make_async_copy(
                tl_hbm.at[pl.ds(0, TK * EMBED)], out_v.at[b], opsems[b],
            ).wait()
        pl.when(blkid(i) < TBLOCKS)(go)

    def transpose(i, b):
        def go():
            @plsc.parallel_loop(0, TK, unroll=8)
            def transpose_col(k):
                col = jnp.zeros((16,), jnp.int32) + k
                lo = plsc.load_gather(in_v.at[b], [row_lo, col])
                hi = plsc.load_gather(in_v.at[b], [row_hi, col])
                base = pl.multiple_of(k * EMBED, EMBED)
                out_v[b, pl.ds(base, 16)] = lo
                out_v[b, pl.ds(base + 16, 16)] = hi
        pl.when(blkid(i) < TBLOCKS)(go)

    @pl.when(wid == NUM_WORKERS - 1)
    def _():
        pltpu.sync_copy(tail_hbm, out_v.at[0, pl.ds(0, TAIL * EMBED)])
        pltpu.sync_copy(out_v.at[0, pl.ds(0, TAIL * EMBED)],
                        tl_hbm.at[pl.ds(TBLOCKS * TK * EMBED, TAIL * EMBED)])

    start_in(0, 0)
    start_in(1, 1)

    def pair_body(j, _):
        for b in (0, 1):
            i = 2 * j + b
            wait_in(i, b)
            pl.when(j > 0)(lambda: wait_out(i - 2, b))
            transpose(i, b)
            start_in(i + 2, b)
            start_out(i, b)
        return ()

    lax.fori_loop(0, TPW // 2, pair_body, ())
    wait_out(TPW - 2, 0)
    wait_out(TPW - 1, 1)


def _sc_body(seq_hbm, table_hbm, out_hbm, idx_v, rows_v, out_v,
             gsem0, gsem1, osem0, osem1, isem0, isem1):
    cid = lax.axis_index("c")
    sid = lax.axis_index("s")
    wid = sid * NUM_CORES + cid
    gsems = (gsem0, gsem1)
    osems = (osem0, osem1)
    isems = (isem0, isem1)

    def stage_idx(g, b):
        base = wid * ROWS_PER_WORKER + g * CHUNK
        pltpu.make_async_copy(
            seq_hbm.at[pl.ds(base, CHUNK)], idx_v.at[b], isems[b],
        ).start()

    def wait_idx(b):
        pltpu.make_async_copy(
            seq_hbm.at[pl.ds(0, CHUNK)], idx_v.at[b], isems[b],
        ).wait()

    def fire_gathers(b):
        for j in range(GATHERS):
            pltpu.make_async_copy(
                table_hbm.at[idx_v.at[b, j // 2, pl.ds(*SPLITS[j % 2])]],
                rows_v.at[b, pl.ds((j // 2) * SEQ_LEN + SPLITS[j % 2][0],
                                   SPLITS[j % 2][1])],
                gsems[b],
            ).start()

    def drain_gathers(b):
        for j in range(GATHERS):
            pltpu.make_async_copy(
                table_hbm.at[idx_v.at[b, j // 2, pl.ds(*SPLITS[j % 2])]],
                rows_v.at[b, pl.ds((j // 2) * SEQ_LEN + SPLITS[j % 2][0],
                                   SPLITS[j % 2][1])],
                gsems[b],
            ).wait()

    def wait_outcopy(b):
        pltpu.make_async_copy(
            out_hbm.at[pl.ds(0, CHUNK)], out_v.at[b], osems[b],
        ).wait()

    def reduce_chunk(g, b):
        for r in range(CHUNK):
            row0 = r * SEQ_LEN
            zero = jnp.zeros((16,), jnp.float32)

            @plsc.parallel_loop(0, SEQ_LEN, step=4, unroll=2,
                                carry=(zero,) * 8)
            def accum(l, carry, row0=row0, b=b):
                a = list(carry)
                for u in range(4):
                    a[u] = a[u] + rows_v[b, row0 + l + u, pl.ds(0, 16)]
                    a[4 + u] = a[4 + u] + rows_v[b, row0 + l + u,
                                                 pl.ds(16, 16)]
                return tuple(a)

            a = accum
            out_v[b, r, pl.ds(0, 16)] = (a[0] + a[1]) + (a[2] + a[3])
            out_v[b, r, pl.ds(16, 16)] = (a[4] + a[5]) + (a[6] + a[7])
        pltpu.make_async_copy(
            out_v.at[b],
            out_hbm.at[pl.ds(wid * ROWS_PER_WORKER + g * CHUNK, CHUNK)],
            osems[b],
        ).start()

    stage_idx(0, 0)
    wait_idx(0)
    fire_gathers(0)
    stage_idx(1, 1)

    def pair_body(k, _):
        g = 2 * k
        wait_idx(1)
        fire_gathers(1)
        drain_gathers(0)
        pl.when(k < NCHUNKS // 2 - 1)(lambda: stage_idx(g + 2, 0))
        pl.when(k > 0)(lambda: wait_outcopy(0))
        reduce_chunk(g, 0)

        def fire_next():
            wait_idx(0)
            fire_gathers(0)
        pl.when(k < NCHUNKS // 2 - 1)(fire_next)
        drain_gathers(1)
        pl.when(k < NCHUNKS // 2 - 1)(lambda: stage_idx(g + 3, 1))
        pl.when(k > 0)(lambda: wait_outcopy(1))
        reduce_chunk(g + 1, 1)
        return ()

    lax.fori_loop(0, NCHUNKS // 2, pair_body, ())
    wait_outcopy(0)
    wait_outcopy(1)


@jax.jit
def kernel(seq, table):
    mesh = plsc.VectorSubcoreMesh(core_axis_name="c", subcore_axis_name="s")
    tp = pl.kernel(
        _tp_body,
        out_type=jax.ShapeDtypeStruct((VOCAB * EMBED,), jnp.float32),
        mesh=mesh,
        scratch_types=[
            pltpu.VMEM((2, EMBED, TK), jnp.float32),
            pltpu.VMEM((2, TK * EMBED), jnp.float32),
            pltpu.SemaphoreType.DMA,
            pltpu.SemaphoreType.DMA,
            pltpu.SemaphoreType.DMA,
            pltpu.SemaphoreType.DMA,
        ],
        compiler_params=pltpu.CompilerParams(use_tc_tiling_on_sc=True,
                                             needs_layout_passes=False),
    )
    tail = table[TBLOCKS * TK:, :].reshape(TAIL * EMBED)
    table = tp(table.T, tail).reshape(VOCAB, EMBED)
    f = pl.kernel(
        _sc_body,
        out_type=jax.ShapeDtypeStruct((BATCH, EMBED), jnp.float32),
        mesh=mesh,
        scratch_types=[
            pltpu.VMEM((2, CHUNK, SEQ_LEN), jnp.int32),
            pltpu.VMEM((2, CHUNK * SEQ_LEN, EMBED), jnp.float32),
            pltpu.VMEM((2, CHUNK, EMBED), jnp.float32),
            pltpu.SemaphoreType.DMA,
            pltpu.SemaphoreType.DMA,
            pltpu.SemaphoreType.DMA,
            pltpu.SemaphoreType.DMA,
            pltpu.SemaphoreType.DMA,
            pltpu.SemaphoreType.DMA,
        ],
        compiler_params=pltpu.CompilerParams(use_tc_tiling_on_sc=False),
    )
    return f(seq, table)


# speedup vs baseline: 1.0600x; 1.0600x over previous
import jax
import jax.numpy as jnp
from jax import lax
from jax.experimental import pallas as pl
from jax.experimental.pallas import tpu as pltpu
from jax.experimental.pallas import tpu_sc as plsc

VOCAB = 1000000
EMBED = 32
BATCH = 16384
SEQ_LEN = 200

NUM_CORES = 2
NUM_SUBCORES = 16
NUM_WORKERS = NUM_CORES * NUM_SUBCORES
ROWS_PER_WORKER = BATCH // NUM_WORKERS
CHUNK = 8
NCHUNKS = ROWS_PER_WORKER // CHUNK
SPLITS = ((0, 104), (104, 96))
GATHERS = len(SPLITS) * CHUNK


TK = 512
TBLOCKS = 999936 // TK
TAIL = VOCAB - TBLOCKS * TK
TPW = -(-TBLOCKS // NUM_WORKERS)


def _tp_body(tt_hbm, tail_hbm, tl_hbm, in_v, out_v, ip0, ip1, op0, op1):
    cid = lax.axis_index("c")
    sid = lax.axis_index("s")
    wid = sid * NUM_CORES + cid
    ipsems = (ip0, ip1)
    opsems = (op0, op1)
    col_stride = jnp.arange(16, dtype=jnp.int32) * EMBED

    def blkid(i):
        return i * NUM_WORKERS + wid

    def start_in(i, b):
        def go():
            pltpu.make_async_copy(
                tt_hbm.at[:, pl.ds(blkid(i) * TK, TK)],
                in_v.at[pl.ds(b * EMBED, EMBED)], ipsems[b],
            ).start()
        pl.when(blkid(i) < TBLOCKS)(go)

    def wait_in(i, b):
        def go():
            pltpu.make_async_copy(
                tt_hbm.at[:, pl.ds(0, TK)],
                in_v.at[pl.ds(b * EMBED, EMBED)], ipsems[b],
            ).wait()
        pl.when(blkid(i) < TBLOCKS)(go)

    def start_out(i, b):
        def go():
            pltpu.make_async_copy(
                out_v.at[pl.ds(b * TK * EMBED, TK * EMBED)],
                tl_hbm.at[pl.ds(blkid(i) * TK * EMBED, TK * EMBED)],
                opsems[b],
            ).start()
        pl.when(blkid(i) < TBLOCKS)(go)

    def wait_out(i, b):
        def go():
            pltpu.make_async_copy(
                tl_hbm.at[pl.ds(0, TK * EMBED)],
                out_v.at[pl.ds(b * TK * EMBED, TK * EMBED)], opsems[b],
            ).wait()
        pl.when(blkid(i) < TBLOCKS)(go)

    def transpose(i, b):
        def go():
            @plsc.parallel_loop(0, TK // 16, unroll=2)
            def tcol(kk):
                off = pl.multiple_of(kk * 16, 16)
                base = kk * (16 * EMBED)
                for e in range(EMBED):
                    v = in_v[b * EMBED + e, pl.ds(off, 16)]
                    plsc.store_scatter(
                        out_v, [col_stride + (b * TK * EMBED + base + e)], v)
        pl.when(blkid(i) < TBLOCKS)(go)

    @pl.when(wid == NUM_WORKERS - 1)
    def _():
        pltpu.sync_copy(tail_hbm, out_v.at[pl.ds(0, TAIL * EMBED)])
        pltpu.sync_copy(out_v.at[pl.ds(0, TAIL * EMBED)],
                        tl_hbm.at[pl.ds(TBLOCKS * TK * EMBED, TAIL * EMBED)])

    start_in(0, 0)
    start_in(1, 1)

    def pair_body(j, _):
        for b in (0, 1):
            i = 2 * j + b
            wait_in(i, b)
            pl.when(j > 0)(lambda: wait_out(i - 2, b))
            transpose(i, b)
            start_in(i + 2, b)
            start_out(i, b)
        return ()

    lax.fori_loop(0, TPW // 2, pair_body, ())
    wait_out(TPW - 2, 0)
    wait_out(TPW - 1, 1)


def _sc_body(seq_hbm, table_hbm, out_hbm, idx_v, rows_v, out_v,
             gsem0, gsem1, osem0, osem1, isem0, isem1):
    cid = lax.axis_index("c")
    sid = lax.axis_index("s")
    wid = sid * NUM_CORES + cid
    gsems = (gsem0, gsem1)
    osems = (osem0, osem1)
    isems = (isem0, isem1)

    def stage_idx(g, b):
        base = wid * ROWS_PER_WORKER + g * CHUNK
        pltpu.make_async_copy(
            seq_hbm.at[pl.ds(base, CHUNK)], idx_v.at[b], isems[b],
        ).start()

    def wait_idx(b):
        pltpu.make_async_copy(
            seq_hbm.at[pl.ds(0, CHUNK)], idx_v.at[b], isems[b],
        ).wait()

    def fire_gathers(b):
        for j in range(GATHERS):
            pltpu.make_async_copy(
                table_hbm.at[idx_v.at[b, j // 2, pl.ds(*SPLITS[j % 2])]],
                rows_v.at[b, pl.ds((j // 2) * SEQ_LEN + SPLITS[j % 2][0],
                                   SPLITS[j % 2][1])],
                gsems[b],
            ).start()

    def drain_gathers(b):
        for j in range(GATHERS):
            pltpu.make_async_copy(
                table_hbm.at[idx_v.at[b, j // 2, pl.ds(*SPLITS[j % 2])]],
                rows_v.at[b, pl.ds((j // 2) * SEQ_LEN + SPLITS[j % 2][0],
                                   SPLITS[j % 2][1])],
                gsems[b],
            ).wait()

    def wait_outcopy(b):
        pltpu.make_async_copy(
            out_hbm.at[pl.ds(0, CHUNK)], out_v.at[b], osems[b],
        ).wait()

    def reduce_chunk(g, b):
        for r in range(CHUNK):
            row0 = r * SEQ_LEN
            zero = jnp.zeros((16,), jnp.float32)

            @plsc.parallel_loop(0, SEQ_LEN, step=4, unroll=2,
                                carry=(zero,) * 8)
            def accum(l, carry, row0=row0, b=b):
                a = list(carry)
                for u in range(4):
                    a[u] = a[u] + rows_v[b, row0 + l + u, pl.ds(0, 16)]
                    a[4 + u] = a[4 + u] + rows_v[b, row0 + l + u,
                                                 pl.ds(16, 16)]
                return tuple(a)

            a = accum
            out_v[b, r, pl.ds(0, 16)] = (a[0] + a[1]) + (a[2] + a[3])
            out_v[b, r, pl.ds(16, 16)] = (a[4] + a[5]) + (a[6] + a[7])
        pltpu.make_async_copy(
            out_v.at[b],
            out_hbm.at[pl.ds(wid * ROWS_PER_WORKER + g * CHUNK, CHUNK)],
            osems[b],
        ).start()

    stage_idx(0, 0)
    wait_idx(0)
    fire_gathers(0)
    stage_idx(1, 1)

    def pair_body(k, _):
        g = 2 * k
        wait_idx(1)
        fire_gathers(1)
        drain_gathers(0)
        pl.when(k < NCHUNKS // 2 - 1)(lambda: stage_idx(g + 2, 0))
        pl.when(k > 0)(lambda: wait_outcopy(0))
        reduce_chunk(g, 0)

        def fire_next():
            wait_idx(0)
            fire_gathers(0)
        pl.when(k < NCHUNKS // 2 - 1)(fire_next)
        drain_gathers(1)
        pl.when(k < NCHUNKS // 2 - 1)(lambda: stage_idx(g + 3, 1))
        pl.when(k > 0)(lambda: wait_outcopy(1))
        reduce_chunk(g + 1, 1)
        return ()

    lax.fori_loop(0, NCHUNKS // 2, pair_body, ())
    wait_outcopy(0)
    wait_outcopy(1)


@jax.jit
def kernel(seq, table):
    mesh = plsc.VectorSubcoreMesh(core_axis_name="c", subcore_axis_name="s")
    tp = pl.kernel(
        _tp_body,
        out_type=jax.ShapeDtypeStruct((VOCAB * EMBED,), jnp.float32),
        mesh=mesh,
        scratch_types=[
            pltpu.VMEM((2 * EMBED, TK), jnp.float32),
            pltpu.VMEM((2 * TK * EMBED,), jnp.float32),
            pltpu.SemaphoreType.DMA,
            pltpu.SemaphoreType.DMA,
            pltpu.SemaphoreType.DMA,
            pltpu.SemaphoreType.DMA,
        ],
        compiler_params=pltpu.CompilerParams(use_tc_tiling_on_sc=True,
                                             needs_layout_passes=False),
    )
    tail = table[TBLOCKS * TK:, :].reshape(TAIL * EMBED)
    table = tp(table.T, tail).reshape(VOCAB, EMBED)
    f = pl.kernel(
        _sc_body,
        out_type=jax.ShapeDtypeStruct((BATCH, EMBED), jnp.float32),
        mesh=mesh,
        scratch_types=[
            pltpu.VMEM((2, CHUNK, SEQ_LEN), jnp.int32),
            pltpu.VMEM((2, CHUNK * SEQ_LEN, EMBED), jnp.float32),
            pltpu.VMEM((2, CHUNK, EMBED), jnp.float32),
            pltpu.SemaphoreType.DMA,
            pltpu.SemaphoreType.DMA,
            pltpu.SemaphoreType.DMA,
            pltpu.SemaphoreType.DMA,
            pltpu.SemaphoreType.DMA,
            pltpu.SemaphoreType.DMA,
        ],
        compiler_params=pltpu.CompilerParams(use_tc_tiling_on_sc=False),
    )
    return f(seq, table)
